# baseline (speedup 1.0000x reference)
import functools

import jax
import jax.numpy as jnp
from jax import lax
from jax.experimental import pallas as pl
from jax.experimental.pallas import tpu as pltpu
from jax.experimental.pallas import tpu_sc as plsc

L = 16
NC = 2
NS = 16
NW = NC * NS


def _build(B=2, D=96, H=384, W=384, N=16384, chunk=64, interpret=False):
    total = B * N
    per_tile = total // NW
    assert per_tile % chunk == 0
    nchunk = per_tile // chunk
    ngrp = per_tile // L
    assert N % per_tile == 0
    assert chunk <= 128

    def body(table, xs, ys, out, xs_v, ys_v,
             idx00, idx10, idx01, idx11,
             w00_v, w10_v, w01_v, w11_v,
             rows0, rows1, out_v, sem):
        cid = lax.axis_index("c")
        sid = lax.axis_index("s")
        wid = sid * NC + cid
        base = wid * per_tile
        row_base = (base // N) * (H * W)

        pltpu.sync_copy(xs.at[pl.ds(base, per_tile)], xs_v)
        pltpu.sync_copy(ys.at[pl.ds(base, per_tile)], ys_v)

        def grp(g, carry):
            sl = pl.ds(g * L, L)
            x = xs_v[sl]
            y = ys_v[sl]
            ix = x - 0.5
            iy = y - 0.5
            x0 = ix.astype(jnp.int32)
            x0 = jnp.where(ix < x0.astype(jnp.float32), x0 - 1, x0)
            y0 = iy.astype(jnp.int32)
            y0 = jnp.where(iy < y0.astype(jnp.float32), y0 - 1, y0)
            wx1 = ix - x0.astype(jnp.float32)
            wx0 = 1.0 - wx1
            wy1 = iy - y0.astype(jnp.float32)
            wy0 = 1.0 - wy1
            wx0 = jnp.where(x0 >= 0, wx0, 0.0)
            wx1 = jnp.where(x0 <= W - 2, wx1, 0.0)
            wy0 = jnp.where(y0 >= 0, wy0, 0.0)
            wy1 = jnp.where(y0 <= H - 2, wy1, 0.0)
            x0c = jnp.maximum(x0, 0)
            x1c = jnp.minimum(x0 + 1, W - 1)
            y0c = jnp.maximum(y0, 0)
            y1c = jnp.minimum(y0 + 1, H - 1)
            r0 = row_base + y0c * W
            r1 = row_base + y1c * W
            c = g // (chunk // L)
            o = (g % (chunk // L)) * L
            csl = pl.ds(o, L)
            idx00[c, csl] = r0 + x0c
            idx10[c, csl] = r0 + x1c
            idx01[c, csl] = r1 + x0c
            idx11[c, csl] = r1 + x1c
            w00_v[sl] = wx0 * wy0
            w10_v[sl] = wx1 * wy0
            w01_v[sl] = wx0 * wy1
            w11_v[sl] = wx1 * wy1
            return carry

        lax.fori_loop(0, ngrp, grp, 0)

        rows = (rows0, rows1)

        def fire(c, buf):
            h0 = pltpu.async_copy(table.at[idx00.at[c]], buf.at[0], sem)
            h1 = pltpu.async_copy(table.at[idx10.at[c]], buf.at[1], sem)
            h2 = pltpu.async_copy(table.at[idx01.at[c]], buf.at[2], sem)
            h3 = pltpu.async_copy(table.at[idx11.at[c]], buf.at[3], sem)
            return (h0, h1, h2, h3)

        def drain(c, buf):
            pltpu.make_async_copy(table.at[idx00.at[c]], buf.at[0], sem).wait()
            pltpu.make_async_copy(table.at[idx10.at[c]], buf.at[1], sem).wait()
            pltpu.make_async_copy(table.at[idx01.at[c]], buf.at[2], sem).wait()
            pltpu.make_async_copy(table.at[idx11.at[c]], buf.at[3], sem).wait()

        def combine(c, buf):
            cbase = c * chunk

            def pt(p, carry2):
                w00 = w00_v[pl.ds(cbase + p, L)][0]
                w10 = w10_v[pl.ds(cbase + p, L)][0]
                w01 = w01_v[pl.ds(cbase + p, L)][0]
                w11 = w11_v[pl.ds(cbase + p, L)][0]
                for j in range(D // L):
                    cs = pl.ds(j * L, L)
                    acc = (w00 * buf[0, p, cs] + w10 * buf[1, p, cs]
                           + w01 * buf[2, p, cs] + w11 * buf[3, p, cs])
                    out_v[p, cs] = acc
                return carry2

            lax.fori_loop(0, chunk, pt, 0)
            pltpu.sync_copy(out_v, out.at[pl.ds(base + cbase, chunk)])

        fire(0, rows0)

        def pair(i2, carry):
            c0 = i2 * 2
            drain(c0, rows0)
            fire(c0 + 1, rows1)
            combine(c0, rows0)
            drain(c0 + 1, rows1)

            @pl.when(c0 + 2 < nchunk)
            def _():
                fire(c0 + 2, rows0)

            combine(c0 + 1, rows1)
            return carry

        lax.fori_loop(0, nchunk // 2, pair, 0)

    mesh = plsc.VectorSubcoreMesh(core_axis_name="c", subcore_axis_name="s",
                                  num_cores=NC, num_subcores=NS)
    return pl.kernel(
        body,
        out_type=jax.ShapeDtypeStruct((total, D), jnp.float32),
        mesh=mesh,
        scratch_types=[
            pltpu.VMEM((per_tile,), jnp.float32),
            pltpu.VMEM((per_tile,), jnp.float32),
            pltpu.VMEM((nchunk, chunk), jnp.int32),
            pltpu.VMEM((nchunk, chunk), jnp.int32),
            pltpu.VMEM((nchunk, chunk), jnp.int32),
            pltpu.VMEM((nchunk, chunk), jnp.int32),
            pltpu.VMEM((per_tile + L,), jnp.float32),
            pltpu.VMEM((per_tile + L,), jnp.float32),
            pltpu.VMEM((per_tile + L,), jnp.float32),
            pltpu.VMEM((per_tile + L,), jnp.float32),
            pltpu.VMEM((4, chunk, D), jnp.float32),
            pltpu.VMEM((4, chunk, D), jnp.float32),
            pltpu.VMEM((chunk, D), jnp.float32),
            pltpu.SemaphoreType.DMA,
        ],
        compiler_params=pltpu.CompilerParams(use_tc_tiling_on_sc=False),
        interpret=interpret,
    )


_sampler = _build()


@jax.jit
def kernel(feature_maps, sample_points):
    B, D, H, W = feature_maps.shape
    N = sample_points.shape[1]
    table = jnp.transpose(feature_maps, (0, 2, 3, 1)).reshape(B * H * W, D)
    xs = sample_points[..., 0].reshape(-1)
    ys = sample_points[..., 1].reshape(-1)
    out = _sampler(table, xs, ys)
    return out.reshape(B, N, D)
